# baseline (speedup 1.0000x reference)
import functools
import math

import jax
import jax.numpy as jnp
from jax import lax
from jax.experimental import pallas as pl
from jax.experimental.pallas import tpu as pltpu

N = 10000
E = 320000
D = 128
H = 64
NC = 2
PAE_IN = 16
PAE_H = 128
BN_EPS = 1e-5
BN_SCALE = 1.0 / math.sqrt(1.0 + BN_EPS)


_EW_B = 2000
_EW_SUB = 8


def _ew_body(en_ref, w1_ref, b1_ref, g_ref, be_ref, w2_ref, b2_ref, out_ref):
    g = g_ref[...] * BN_SCALE

    def parser(x):
        h = jnp.dot(x, w1_ref[...], preferred_element_type=jnp.float32) + b1_ref[...]
        h = jnp.maximum(h, 0.0)
        h = h * g + be_ref[...]
        return jnp.dot(h, w2_ref[...], preferred_element_type=jnp.float32) + b2_ref[...]

    for r in range(_EW_SUB):
        z = en_ref[pl.ds(r * _EW_B, _EW_B), :]
        h1 = parser(z[:, :PAE_IN])
        h2 = parser(z[:, PAE_IN:])
        n1 = jnp.maximum(jnp.sqrt(jnp.sum(h1 * h1, axis=1)), 1e-8)
        n2 = jnp.maximum(jnp.sqrt(jnp.sum(h2 * h2, axis=1)), 1e-8)
        out_ref[r, :] = (jnp.sum(h1 * h2, axis=1) / (n1 * n2) + 1.0) * 0.5


def _edge_weights(edgenet_input, pae_w1, pae_b1, pae_g, pae_be, pae_w2, pae_b2):
    grid = E // (_EW_B * _EW_SUB)
    out = pl.pallas_call(
        _ew_body,
        grid=(grid,),
        in_specs=[
            pl.BlockSpec((_EW_B * _EW_SUB, 2 * PAE_IN), lambda i: (i, 0)),
            pl.BlockSpec((PAE_IN, PAE_H), lambda i: (0, 0)),
            pl.BlockSpec((PAE_H,), lambda i: (0,)),
            pl.BlockSpec((PAE_H,), lambda i: (0,)),
            pl.BlockSpec((PAE_H,), lambda i: (0,)),
            pl.BlockSpec((PAE_H, PAE_H), lambda i: (0, 0)),
            pl.BlockSpec((PAE_H,), lambda i: (0,)),
        ],
        out_specs=pl.BlockSpec((_EW_SUB, _EW_B), lambda i: (i, 0)),
        out_shape=jax.ShapeDtypeStruct((E // _EW_B, _EW_B), jnp.float32),
    )(edgenet_input, pae_w1, pae_b1, pae_g, pae_be, pae_w2, pae_b2)
    return out.reshape(E)


def _combine_body(t0_ref, t1_ref, p2_ref, w_ref, out_ref):
    t0 = t0_ref[...]
    t2 = 2.0 * p2_ref[...] - t0
    w = w_ref[...]
    acc = jnp.dot(t0, w[0], preferred_element_type=jnp.float32)
    acc += jnp.dot(t1_ref[...], w[1], preferred_element_type=jnp.float32)
    acc += jnp.dot(t2, w[2], preferred_element_type=jnp.float32)
    out_ref[...] = jnp.maximum(acc, 0.0)


def _cheb_combine(t0, t1, p2, w):
    d_in = t0.shape[1]
    return pl.pallas_call(
        _combine_body,
        grid=(5,),
        in_specs=[
            pl.BlockSpec((2000, d_in), lambda i: (i, 0)),
            pl.BlockSpec((2000, d_in), lambda i: (i, 0)),
            pl.BlockSpec((2000, d_in), lambda i: (i, 0)),
            pl.BlockSpec((3, d_in, H), lambda i: (0, 0, 0)),
        ],
        out_specs=pl.BlockSpec((2000, H), lambda i: (i, 0)),
        out_shape=jax.ShapeDtypeStruct((N, H), jnp.float32),
    )(t0, t1, p2, w)


def _cls_body(h1_ref, h2_ref, h3_ref, h4_ref, w1_ref, b1_ref, g_ref, be_ref,
              w2_ref, b2_ref, out_ref):
    z = jnp.dot(h1_ref[...], w1_ref[0], preferred_element_type=jnp.float32)
    z += jnp.dot(h2_ref[...], w1_ref[1], preferred_element_type=jnp.float32)
    z += jnp.dot(h3_ref[...], w1_ref[2], preferred_element_type=jnp.float32)
    z += jnp.dot(h4_ref[...], w1_ref[3], preferred_element_type=jnp.float32)
    z += b1_ref[...]
    z = jnp.maximum(z, 0.0)
    z = z * (g_ref[...] * BN_SCALE) + be_ref[...]
    out_ref[...] = jnp.dot(z, w2_ref[...], preferred_element_type=jnp.float32) + b2_ref[...]


def _classifier(h1, h2, h3, h4, cls_w1, cls_b1, cls_g, cls_be, cls_w2, cls_b2):
    w1 = cls_w1.reshape(4, H, 256)
    return pl.pallas_call(
        _cls_body,
        grid=(5,),
        in_specs=[
            pl.BlockSpec((2000, H), lambda i: (i, 0)),
            pl.BlockSpec((2000, H), lambda i: (i, 0)),
            pl.BlockSpec((2000, H), lambda i: (i, 0)),
            pl.BlockSpec((2000, H), lambda i: (i, 0)),
            pl.BlockSpec((4, H, 256), lambda i: (0, 0, 0)),
            pl.BlockSpec((256,), lambda i: (0,)),
            pl.BlockSpec((256,), lambda i: (0,)),
            pl.BlockSpec((256,), lambda i: (0,)),
            pl.BlockSpec((256, NC), lambda i: (0, 0)),
            pl.BlockSpec((NC,), lambda i: (0,)),
        ],
        out_specs=pl.BlockSpec((2000, NC), lambda i: (i, 0)),
        out_shape=jax.ShapeDtypeStruct((N, NC), jnp.float32),
    )(h1, h2, h3, h4, w1, cls_b1, cls_g, cls_be, cls_w2, cls_b2)


def _prop(x, row, col, norm):
    return jnp.zeros_like(x).at[col].add(norm[:, None] * x[row])


def kernel(features, edge_index, edgenet_input, pae_w1, pae_b1, pae_g, pae_be,
           pae_w2, pae_b2, cheb0, cheb1, cheb2, cheb3, cls_w1, cls_b1, cls_g,
           cls_be, cls_w2, cls_b2):
    edge_weight = _edge_weights(edgenet_input, pae_w1, pae_b1, pae_g, pae_be,
                                pae_w2, pae_b2)
    row = edge_index[0]
    col = edge_index[1]
    deg = jnp.zeros((N,), jnp.float32).at[row].add(edge_weight)
    dis = jnp.where(deg > 0, lax.rsqrt(jnp.where(deg > 0, deg, 1.0)), 0.0)
    norm = -dis[row] * edge_weight * dis[col]

    hs = []
    h = features
    for w in (cheb0, cheb1, cheb2, cheb3):
        t1 = _prop(h, row, col, norm)
        p2 = _prop(t1, row, col, norm)
        h = _cheb_combine(h, t1, p2, w)
        hs.append(h)

    return _classifier(hs[0], hs[1], hs[2], hs[3], cls_w1, cls_b1, cls_g,
                       cls_be, cls_w2, cls_b2)


# speedup vs baseline: 4.0345x; 4.0345x over previous
import functools
import math

import jax
import jax.numpy as jnp
from jax import lax
from jax.experimental import pallas as pl
from jax.experimental.pallas import tpu as pltpu
from jax.experimental.pallas import tpu_sc as plsc

N = 10000
E = 320000
D = 128
H = 64
NC = 2
PAE_IN = 16
PAE_H = 128
BN_EPS = 1e-5
BN_SCALE = 1.0 / math.sqrt(1.0 + BN_EPS)

_NPAD = 10240
_RPT = 640
_BLK = 128
_NBLK = E // _BLK
_MESH = plsc.VectorSubcoreMesh(core_axis_name="c", subcore_axis_name="s")


_EW_B = 2000
_EW_SUB = 8


def _ew_body(en_ref, w1_ref, b1_ref, g_ref, be_ref, w2_ref, b2_ref, out_ref):
    g = g_ref[...] * BN_SCALE

    def parser(x):
        h = jnp.dot(x, w1_ref[...], preferred_element_type=jnp.float32) + b1_ref[...]
        h = jnp.maximum(h, 0.0)
        h = h * g + be_ref[...]
        return jnp.dot(h, w2_ref[...], preferred_element_type=jnp.float32) + b2_ref[...]

    for r in range(_EW_SUB):
        z = en_ref[pl.ds(r * _EW_B, _EW_B), :]
        h1 = parser(z[:, :PAE_IN])
        h2 = parser(z[:, PAE_IN:])
        n1 = jnp.maximum(jnp.sqrt(jnp.sum(h1 * h1, axis=1)), 1e-8)
        n2 = jnp.maximum(jnp.sqrt(jnp.sum(h2 * h2, axis=1)), 1e-8)
        out_ref[r, :] = (jnp.sum(h1 * h2, axis=1) / (n1 * n2) + 1.0) * 0.5


def _edge_weights(edgenet_input, pae_w1, pae_b1, pae_g, pae_be, pae_w2, pae_b2):
    grid = E // (_EW_B * _EW_SUB)
    out = pl.pallas_call(
        _ew_body,
        grid=(grid,),
        in_specs=[
            pl.BlockSpec((_EW_B * _EW_SUB, 2 * PAE_IN), lambda i: (i, 0)),
            pl.BlockSpec((PAE_IN, PAE_H), lambda i: (0, 0)),
            pl.BlockSpec((PAE_H,), lambda i: (0,)),
            pl.BlockSpec((PAE_H,), lambda i: (0,)),
            pl.BlockSpec((PAE_H,), lambda i: (0,)),
            pl.BlockSpec((PAE_H, PAE_H), lambda i: (0, 0)),
            pl.BlockSpec((PAE_H,), lambda i: (0,)),
        ],
        out_specs=pl.BlockSpec((_EW_SUB, _EW_B), lambda i: (i, 0)),
        out_shape=jax.ShapeDtypeStruct((E // _EW_B, _EW_B), jnp.float32),
    )(edgenet_input, pae_w1, pae_b1, pae_g, pae_be, pae_w2, pae_b2)
    return out.reshape(_NBLK, _BLK)


def _rsqrt_nr(d):
    i = lax.bitcast_convert_type(d, jnp.int32)
    i = jnp.int32(0x5F3759DF) - (i >> 1)
    y = lax.bitcast_convert_type(i, jnp.float32)
    for _ in range(4):
        y = y * (1.5 - 0.5 * d * y * y)
    return jnp.where(d > 0.0, y, 0.0)


def _worker_span(nblk, nworkers, wid):
    q, r = divmod(nblk, nworkers)
    cnt = q + jnp.where(wid < r, 1, 0)
    start = q * wid + jnp.minimum(wid, r)
    return start, cnt


def _norm_body(row_hbm, col_hbm, ew_hbm, out_hbm,
               deg_sh, dis_sh, idx_v, cid_v, val_v, nrm_v, dis_v, dz_v):
    cid = lax.axis_index("c")
    sid = lax.axis_index("s")
    wid = sid * 2 + cid

    for j in range(_RPT // 16):
        dz_v[pl.ds(j * 16, 16)] = jnp.zeros((16,), jnp.float32)
    pltpu.sync_copy(dz_v, deg_sh.at[pl.ds(sid * _RPT, _RPT)])
    plsc.subcore_barrier()

    start_a, cnt_a = _worker_span(_NBLK, 16, sid)

    def blk_a(i, _):
        b = start_a + i
        pltpu.sync_copy(row_hbm.at[b], idx_v)
        pltpu.sync_copy(ew_hbm.at[b], val_v)
        pltpu.sync_copy(val_v, deg_sh.at[idx_v], add=True)
        return 0

    lax.fori_loop(0, cnt_a, blk_a, 0)
    plsc.subcore_barrier()

    pltpu.sync_copy(deg_sh.at[pl.ds(sid * _RPT, _RPT)], dz_v)
    for j in range(_RPT // 16):
        dz_v[pl.ds(j * 16, 16)] = _rsqrt_nr(dz_v[pl.ds(j * 16, 16)])
    pltpu.sync_copy(dz_v, dis_sh.at[pl.ds(sid * _RPT, _RPT)])
    plsc.subcore_barrier()

    pltpu.sync_copy(dis_sh, dis_v)
    start_c, cnt_c = _worker_span(_NBLK, 32, wid)

    def blk_c(i, _):
        b = start_c + i
        pltpu.sync_copy(row_hbm.at[b], idx_v)
        pltpu.sync_copy(col_hbm.at[b], cid_v)
        pltpu.sync_copy(ew_hbm.at[b], val_v)
        for j in range(_BLK // 16):
            sl = pl.ds(j * 16, 16)
            dr = plsc.load_gather(dis_v, [idx_v[sl]])
            dc = plsc.load_gather(dis_v, [cid_v[sl]])
            nrm_v[sl] = -(dr * val_v[sl] * dc)
        pltpu.sync_copy(nrm_v, out_hbm.at[b])
        return 0

    lax.fori_loop(0, cnt_c, blk_c, 0)


_norm_sc = pl.kernel(
    _norm_body,
    out_type=jax.ShapeDtypeStruct((_NBLK, _BLK), jnp.float32),
    mesh=_MESH,
    compiler_params=pltpu.CompilerParams(needs_layout_passes=False),
    scratch_types=[
        pltpu.MemorySpace.VMEM_SHARED((_NPAD,), jnp.float32),
        pltpu.MemorySpace.VMEM_SHARED((_NPAD,), jnp.float32),
        pltpu.VMEM((_BLK,), jnp.int32),
        pltpu.VMEM((_BLK,), jnp.int32),
        pltpu.VMEM((_BLK,), jnp.float32),
        pltpu.VMEM((_BLK,), jnp.float32),
        pltpu.VMEM((_NPAD,), jnp.float32),
        pltpu.VMEM((_RPT,), jnp.float32),
    ],
)


def _prop_body(dx, x_hbm, row_hbm, col_hbm, nrm_hbm, out_hbm,
               acc_sh, idx_v, cid_v, nrm_v, rows_v, zz_v, sem):
    cid = lax.axis_index("c")
    sid = lax.axis_index("s")
    wid = sid * 2 + cid

    def zr(r, _):
        for j in range(dx // 16):
            zz_v[r, pl.ds(j * 16, 16)] = jnp.zeros((16,), jnp.float32)
        return 0

    lax.fori_loop(0, 64, zr, 0)
    for t in range(_RPT // 64):
        pltpu.sync_copy(zz_v, acc_sh.at[pl.ds(sid * _RPT + t * 64, 64)])
    plsc.subcore_barrier()

    start, cnt = _worker_span(_NBLK, 32, wid)

    def blk(i, _):
        b = start + i
        pltpu.sync_copy(row_hbm.at[b], idx_v)
        pltpu.sync_copy(col_hbm.at[b], cid_v)
        pltpu.sync_copy(nrm_hbm.at[b], nrm_v)
        pltpu.async_copy(x_hbm.at[idx_v], rows_v, sem).wait()

        def edge(e, _):
            s = plsc.load_gather(nrm_v, [jnp.full((16,), e, jnp.int32)])
            for j in range(dx // 16):
                sl = pl.ds(j * 16, 16)
                rows_v[e, sl] = rows_v[e, sl] * s
            return 0

        lax.fori_loop(0, _BLK, edge, 0)
        pltpu.sync_copy(rows_v, acc_sh.at[cid_v], add=True)
        return 0

    lax.fori_loop(0, cnt, blk, 0)
    plsc.subcore_barrier()
    pltpu.sync_copy(acc_sh.at[pl.ds(sid * _RPT, _RPT)],
                    out_hbm.at[cid, pl.ds(sid * _RPT, _RPT)])


_prop = pl.kernel(
    functools.partial(_prop_body, D),
    out_type=jax.ShapeDtypeStruct((2, _NPAD, D), jnp.float32),
    mesh=_MESH,
    compiler_params=pltpu.CompilerParams(needs_layout_passes=False),
    scratch_types=[
        pltpu.MemorySpace.VMEM_SHARED((_NPAD, D), jnp.float32),
        pltpu.VMEM((_BLK,), jnp.int32),
        pltpu.VMEM((_BLK,), jnp.int32),
        pltpu.VMEM((_BLK,), jnp.float32),
        pltpu.VMEM((_BLK, D), jnp.float32),
        pltpu.VMEM((64, D), jnp.float32),
        pltpu.SemaphoreType.DMA,
    ],
)


def _psum_body(p_ref, o_ref):
    o_ref[...] = p_ref[0] + p_ref[1]


def _pair_sum(p):
    return pl.pallas_call(
        _psum_body,
        grid=(5,),
        in_specs=[pl.BlockSpec((2, 2000, D), lambda i: (0, i, 0))],
        out_specs=pl.BlockSpec((2000, D), lambda i: (i, 0)),
        out_shape=jax.ShapeDtypeStruct((N, D), jnp.float32),
    )(p)


def _combine_body(d_in, t0_ref, t1_ref, p2_ref, w_ref, out_ref):
    t0 = t0_ref[:, :d_in]
    t1 = t1_ref[:, :d_in]
    t2 = 2.0 * (p2_ref[0, :, :d_in] + p2_ref[1, :, :d_in]) - t0
    w = w_ref[...]
    acc = jnp.dot(t0, w[0], preferred_element_type=jnp.float32)
    acc += jnp.dot(t1, w[1], preferred_element_type=jnp.float32)
    acc += jnp.dot(t2, w[2], preferred_element_type=jnp.float32)
    out_ref[:, :H] = jnp.maximum(acc, 0.0)
    out_ref[:, H:] = jnp.zeros((out_ref.shape[0], D - H), jnp.float32)


def _cheb_combine(t0, t1, p2, w):
    d_in = w.shape[1]
    return pl.pallas_call(
        functools.partial(_combine_body, d_in),
        grid=(5,),
        in_specs=[
            pl.BlockSpec((2000, D), lambda i: (i, 0)),
            pl.BlockSpec((2000, D), lambda i: (i, 0)),
            pl.BlockSpec((2, 2000, D), lambda i: (0, i, 0)),
            pl.BlockSpec((3, d_in, H), lambda i: (0, 0, 0)),
        ],
        out_specs=pl.BlockSpec((2000, D), lambda i: (i, 0)),
        out_shape=jax.ShapeDtypeStruct((N, D), jnp.float32),
    )(t0, t1, p2, w)


def _cls_body(h1_ref, h2_ref, h3_ref, h4_ref, w1_ref, b1_ref, g_ref, be_ref,
              w2_ref, b2_ref, out_ref):
    z = jnp.dot(h1_ref[:, :H], w1_ref[0], preferred_element_type=jnp.float32)
    z += jnp.dot(h2_ref[:, :H], w1_ref[1], preferred_element_type=jnp.float32)
    z += jnp.dot(h3_ref[:, :H], w1_ref[2], preferred_element_type=jnp.float32)
    z += jnp.dot(h4_ref[:, :H], w1_ref[3], preferred_element_type=jnp.float32)
    z += b1_ref[...]
    z = jnp.maximum(z, 0.0)
    z = z * (g_ref[...] * BN_SCALE) + be_ref[...]
    out_ref[...] = jnp.dot(z, w2_ref[...], preferred_element_type=jnp.float32) + b2_ref[...]


def _classifier(h1, h2, h3, h4, cls_w1, cls_b1, cls_g, cls_be, cls_w2, cls_b2):
    w1 = cls_w1.reshape(4, H, 256)
    return pl.pallas_call(
        _cls_body,
        grid=(5,),
        in_specs=[
            pl.BlockSpec((2000, D), lambda i: (i, 0)),
            pl.BlockSpec((2000, D), lambda i: (i, 0)),
            pl.BlockSpec((2000, D), lambda i: (i, 0)),
            pl.BlockSpec((2000, D), lambda i: (i, 0)),
            pl.BlockSpec((4, H, 256), lambda i: (0, 0, 0)),
            pl.BlockSpec((256,), lambda i: (0,)),
            pl.BlockSpec((256,), lambda i: (0,)),
            pl.BlockSpec((256,), lambda i: (0,)),
            pl.BlockSpec((256, NC), lambda i: (0, 0)),
            pl.BlockSpec((NC,), lambda i: (0,)),
        ],
        out_specs=pl.BlockSpec((2000, NC), lambda i: (i, 0)),
        out_shape=jax.ShapeDtypeStruct((N, NC), jnp.float32),
    )(h1, h2, h3, h4, w1, cls_b1, cls_g, cls_be, cls_w2, cls_b2)


def kernel(features, edge_index, edgenet_input, pae_w1, pae_b1, pae_g, pae_be,
           pae_w2, pae_b2, cheb0, cheb1, cheb2, cheb3, cls_w1, cls_b1, cls_g,
           cls_be, cls_w2, cls_b2):
    ew2 = _edge_weights(edgenet_input, pae_w1, pae_b1, pae_g, pae_be,
                        pae_w2, pae_b2)
    row2 = edge_index[0].reshape(_NBLK, _BLK)
    col2 = edge_index[1].reshape(_NBLK, _BLK)
    norm2 = _norm_sc(row2, col2, ew2)

    hs = []
    h = features
    for w in (cheb0, cheb1, cheb2, cheb3):
        p1 = _prop(h, row2, col2, norm2)
        t1 = _pair_sum(p1)
        p2 = _prop(t1, row2, col2, norm2)
        h = _cheb_combine(h, t1, p2, w)
        hs.append(h)

    return _classifier(hs[0], hs[1], hs[2], hs[3], cls_w1, cls_b1, cls_g,
                       cls_be, cls_w2, cls_b2)
